# Initial kernel scaffold; baseline (speedup 1.0000x reference)
import jax
import jax.numpy as jnp
from jax.experimental import pallas as pl


def kernel(x, edge_index, W1, b1, W2, b2, W3, b3, g1, be1, g2, be2, pw1, pb1, pw2, pb2):
    raise NotImplementedError("write your pallas kernel here")


# speedup vs baseline: 3.9956x; 3.9956x over previous
import functools

import jax
import jax.numpy as jnp
from jax import lax
from jax.experimental import pallas as pl
from jax.experimental.pallas import tpu as pltpu
from jax.experimental.pallas import tpu_sc as plsc

N = 100000
E = 1600000
D_IN = 128
D_H = 32
D_OUT = 16

NC = 2
NS = 16
LANE = 16
BATCH = 128
K = 8

QUARTER = N // 4
QR = 25600
NQ = 4
HALF = N // 2
DR = 51200
E_PAD = 1605632
ROWS2D = E_PAD // BATCH
RPT = ROWS2D // NS
STEPS = RPT // K


NSUB = BATCH // LANE


def _filter_chunk(src_st, dst_st, csrc, cdst2, base, bound):
    cnt = jnp.int32(0)
    one16 = jnp.ones((LANE,), jnp.int32)
    for j in range(K):
        for l in range(NSUB):
            d = dst_st[j, pl.ds(l * LANE, LANE)]
            vl = d - base
            ok = (vl >= 0) & (vl < bound)
            pos = cnt + plsc.cumsum(one16, mask=ok) - 1
            plsc.store_scatter(cdst2, [pos >> 7, pos & (BATCH - 1)], vl,
                               mask=ok)
            if csrc is not None:
                sidx = src_st[j, pl.ds(l * LANE, LANE)]
                plsc.store_scatter(csrc, [pos], sidx, mask=ok)
            cnt = cnt + plsc.all_reduce_population_count(ok)[0]
    nb = (cnt + (BATCH - 1)) // BATCH
    end = nb * BATCH
    dummyv = jnp.full((LANE,), bound, jnp.int32)
    zerov = jnp.zeros((LANE,), jnp.int32)
    iota = lax.iota(jnp.int32, LANE)
    for t in range(NSUB):
        pos = cnt + t * LANE + iota
        m = pos < end
        plsc.store_scatter(cdst2, [pos >> 7, pos & (BATCH - 1)], dummyv,
                           mask=m)
        if csrc is not None:
            plsc.store_scatter(csrc, [pos], zerov, mask=m)
    return nb


def _sc_agg_body(p_hbm, src_hbm, dst_hbm, zeros_hbm, out_hbm,
                 src_st0, dst_st0, csrc0, cdst20, rows0,
                 src_st1, dst_st1, csrc1, cdst21, rows1,
                 sem_i, sem_g0, sem_g1, sem_s0, sem_s1, acc):
    c = lax.axis_index("c")
    s = lax.axis_index("s")
    zr_pt = QR // NS
    row0 = s * RPT
    st = [(src_st0, dst_st0, csrc0, None, cdst20, rows0, sem_g0, sem_s0),
          (src_st1, dst_st1, csrc1, None, cdst21, rows1, sem_g1, sem_s1)]

    def fire_idx(r, b):
        pltpu.async_copy(src_hbm.at[pl.ds(r, K)], st[b][0], sem_i)
        pltpu.async_copy(dst_hbm.at[pl.ds(r, K)], st[b][1], sem_i)

    def wait_idx(r, b):
        pltpu.make_async_copy(src_hbm.at[pl.ds(r, K)], st[b][0], sem_i).wait()
        pltpu.make_async_copy(dst_hbm.at[pl.ds(r, K)], st[b][1], sem_i).wait()

    def fire_gathers(b, nb):
        for j in range(K):
            @pl.when(j < nb)
            def _():
                pltpu.async_copy(
                    p_hbm.at[st[b][2].at[pl.ds(j * BATCH, BATCH)]],
                    st[b][5].at[pl.ds(j * BATCH, BATCH)], st[b][6])

    def wait_gathers(b, nb):
        for j in range(K):
            @pl.when(j < nb)
            def _():
                pltpu.make_async_copy(
                    p_hbm.at[st[b][2].at[pl.ds(j * BATCH, BATCH)]],
                    st[b][5].at[pl.ds(j * BATCH, BATCH)], st[b][6]).wait()

    def fire_scatters(b, nb):
        for j in range(K):
            @pl.when(j < nb)
            def _():
                pltpu.async_copy(st[b][5].at[pl.ds(j * BATCH, BATCH)],
                                 acc.at[st[b][4].at[j]], st[b][7], add=True)

    def wait_scatters(b, nb):
        for j in range(K):
            @pl.when(j < nb)
            def _():
                pltpu.make_async_copy(st[b][5].at[pl.ds(j * BATCH, BATCH)],
                                      acc.at[st[b][4].at[j]], st[b][7]).wait()

    def filt(b, base):
        return _filter_chunk(st[b][0], st[b][1], st[b][2],
                             st[b][4], base, QUARTER)

    def qpass(q, qcarry):
        pltpu.sync_copy(zeros_hbm.at[pl.ds(s * zr_pt, zr_pt)],
                        acc.at[pl.ds(s * zr_pt, zr_pt)])
        plsc.subcore_barrier()

        base = (c * 2 + q) * QUARTER

        pltpu.sync_copy(src_hbm.at[pl.ds(row0, K)], src_st0)
        pltpu.sync_copy(dst_hbm.at[pl.ds(row0, K)], dst_st0)
        nb0 = filt(0, base)
        fire_gathers(0, nb0)
        fire_idx(row0 + K, 1)
        wait_gathers(0, nb0)
        fire_scatters(0, nb0)
        wait_idx(row0 + K, 1)
        nb1 = filt(1, base)
        fire_gathers(1, nb1)

        def stage(cur, cth, nxt, nb_cur, nb_old):
            fire_idx(row0 + (cth + 1) * K, nxt)
            wait_gathers(cur, nb_cur)
            fire_scatters(cur, nb_cur)
            wait_scatters(nxt, nb_old)
            wait_idx(row0 + (cth + 1) * K, nxt)
            nb_new = filt(nxt, base)
            fire_gathers(nxt, nb_new)
            return nb_new

        def pair(ii, carry):
            na, nb = carry
            a = 2 * ii + 1
            na2 = stage(1, a, 0, nb, na)
            nb2 = stage(0, a + 1, 1, na2, nb)
            return (na2, nb2)

        nb0, nb1 = lax.fori_loop(0, (STEPS - 2) // 2, pair, (nb0, nb1))
        wait_gathers(1, nb1)
        fire_scatters(1, nb1)
        wait_scatters(0, nb0)
        wait_scatters(1, nb1)

        plsc.subcore_barrier()
        pltpu.sync_copy(acc.at[pl.ds(s * zr_pt, zr_pt)],
                        out_hbm.at[pl.ds((c * 2 + q) * QR + s * zr_pt, zr_pt)])
        plsc.subcore_barrier()
        return qcarry

    lax.fori_loop(0, 2, qpass, 0)


@functools.cache
def _sc_agg_kernel():
    return pl.kernel(
        _sc_agg_body,
        out_type=jax.ShapeDtypeStruct((NQ * QR, D_H), jnp.float32),
        mesh=plsc.VectorSubcoreMesh(core_axis_name="c", subcore_axis_name="s",
                                    num_cores=NC, num_subcores=NS),
        scratch_types=[
            pltpu.VMEM((K, BATCH), jnp.int32),
            pltpu.VMEM((K, BATCH), jnp.int32),
            pltpu.VMEM((K * BATCH,), jnp.int32),
            pltpu.VMEM((K, BATCH), jnp.int32),
            pltpu.VMEM((K * BATCH, D_H), jnp.float32),
            pltpu.VMEM((K, BATCH), jnp.int32),
            pltpu.VMEM((K, BATCH), jnp.int32),
            pltpu.VMEM((K * BATCH,), jnp.int32),
            pltpu.VMEM((K, BATCH), jnp.int32),
            pltpu.VMEM((K * BATCH, D_H), jnp.float32),
            pltpu.SemaphoreType.DMA,
            pltpu.SemaphoreType.DMA,
            pltpu.SemaphoreType.DMA,
            pltpu.SemaphoreType.DMA,
            pltpu.SemaphoreType.DMA,
            pltpu.VMEM_SHARED((QR, D_H), jnp.float32),
        ],
        compiler_params=pltpu.CompilerParams(use_tc_tiling_on_sc=False,
                                             internal_scratch_in_bytes=65536,
                                             needs_layout_passes=False),
    )


def _sc_deg_body(dst_hbm, zeros_hbm, out_hbm,
                 dst_st0, cdst20, dst_st1, cdst21, ones,
                 sem_i, sem_s0, sem_s1, acc):
    c = lax.axis_index("c")
    s = lax.axis_index("s")
    dr_pt = DR // NS
    row0 = s * RPT
    st = [(dst_st0, None, cdst20, sem_s0), (dst_st1, None, cdst21, sem_s1)]
    for l in range(NSUB):
        ones[pl.ds(l * LANE, LANE)] = jnp.ones((LANE,), jnp.float32)

    def fire_scatters(b, nb):
        for j in range(K):
            @pl.when(j < nb)
            def _():
                pltpu.async_copy(ones, acc.at[st[b][2].at[j]], st[b][3],
                                 add=True)

    def wait_scatters(b, nb):
        for j in range(K):
            @pl.when(j < nb)
            def _():
                pltpu.make_async_copy(ones, acc.at[st[b][2].at[j]],
                                      st[b][3]).wait()

    def filt(b, base):
        return _filter_chunk(None, st[b][0], None, st[b][2], base, HALF)

    pltpu.sync_copy(zeros_hbm.at[pl.ds(s * dr_pt, dr_pt)],
                    acc.at[pl.ds(s * dr_pt, dr_pt)])
    plsc.subcore_barrier()

    base = c * HALF

    pltpu.sync_copy(dst_hbm.at[pl.ds(row0, K)], dst_st0)
    nb0 = filt(0, base)
    pltpu.async_copy(dst_hbm.at[pl.ds(row0 + K, K)], dst_st1, sem_i)
    fire_scatters(0, nb0)
    pltpu.make_async_copy(dst_hbm.at[pl.ds(row0 + K, K)], dst_st1,
                          sem_i).wait()
    nb1 = filt(1, base)

    def stage(cur, cth, nxt, nb_cur, nb_old):
        r = row0 + (cth + 1) * K
        pltpu.async_copy(dst_hbm.at[pl.ds(r, K)], st[nxt][0], sem_i)
        fire_scatters(cur, nb_cur)
        wait_scatters(nxt, nb_old)
        pltpu.make_async_copy(dst_hbm.at[pl.ds(r, K)], st[nxt][0],
                              sem_i).wait()
        return filt(nxt, base)

    def pair(ii, carry):
        na, nb = carry
        a = 2 * ii + 1
        na2 = stage(1, a, 0, nb, na)
        nb2 = stage(0, a + 1, 1, na2, nb)
        return (na2, nb2)

    nb0, nb1 = lax.fori_loop(0, (STEPS - 2) // 2, pair, (nb0, nb1))
    fire_scatters(1, nb1)
    wait_scatters(0, nb0)
    wait_scatters(1, nb1)

    plsc.subcore_barrier()
    pltpu.sync_copy(acc.at[pl.ds(s * dr_pt, dr_pt)],
                    out_hbm.at[pl.ds(c * DR + s * dr_pt, dr_pt)])


@functools.cache
def _sc_deg_kernel():
    return pl.kernel(
        _sc_deg_body,
        out_type=jax.ShapeDtypeStruct((NC * DR,), jnp.float32),
        mesh=plsc.VectorSubcoreMesh(core_axis_name="c", subcore_axis_name="s",
                                    num_cores=NC, num_subcores=NS),
        scratch_types=[
            pltpu.VMEM((K, BATCH), jnp.int32),
            pltpu.VMEM((K, BATCH), jnp.int32),
            pltpu.VMEM((K, BATCH), jnp.int32),
            pltpu.VMEM((K, BATCH), jnp.int32),
            pltpu.VMEM((BATCH,), jnp.float32),
            pltpu.SemaphoreType.DMA,
            pltpu.SemaphoreType.DMA,
            pltpu.SemaphoreType.DMA,
            pltpu.VMEM_SHARED((DR,), jnp.float32),
        ],
        compiler_params=pltpu.CompilerParams(use_tc_tiling_on_sc=False,
                                             internal_scratch_in_bytes=65536,
                                             needs_layout_passes=False),
    )


_R = 2000
_GRID = N // _R


def _tc_prep_body(x_ref, deg_ref, w_ref, h_ref, p_ref):
    h = jnp.dot(x_ref[...], w_ref[...], preferred_element_type=jnp.float32)
    dinv = lax.rsqrt(deg_ref[...] + 1.0)
    h_ref[...] = h
    p_ref[...] = h * dinv


_tc_prep = pl.pallas_call(
    _tc_prep_body,
    grid=(_GRID,),
    in_specs=[
        pl.BlockSpec((_R, D_IN), lambda i: (i, 0)),
        pl.BlockSpec((_R, 1), lambda i: (i, 0)),
        pl.BlockSpec((D_IN, D_H), lambda i: (0, 0)),
    ],
    out_specs=[pl.BlockSpec((_R, D_H), lambda i: (i, 0))] * 2,
    out_shape=[jax.ShapeDtypeStruct((N, D_H), jnp.float32)] * 2,
)


def _tc_mid_body(a_ref, h_ref, deg_ref, w_ref, b_ref, g_ref, be_ref,
                 hn_ref, pn_ref):
    dinv = lax.rsqrt(deg_ref[...] + 1.0)
    z = dinv * a_ref[...] + (dinv * dinv) * h_ref[...] + b_ref[...]
    r = jnp.maximum(z, 0.0)
    mu = jnp.mean(r, axis=-1, keepdims=True)
    d = r - mu
    var = jnp.mean(d * d, axis=-1, keepdims=True)
    r = d * lax.rsqrt(var + 1e-5) * g_ref[...] + be_ref[...]
    hn = jnp.dot(r, w_ref[...], preferred_element_type=jnp.float32)
    hn_ref[...] = hn
    pn_ref[...] = hn * dinv


_tc_mid = pl.pallas_call(
    _tc_mid_body,
    grid=(_GRID,),
    in_specs=[
        pl.BlockSpec((_R, D_H), lambda i: (i, 0)),
        pl.BlockSpec((_R, D_H), lambda i: (i, 0)),
        pl.BlockSpec((_R, 1), lambda i: (i, 0)),
        pl.BlockSpec((D_H, D_H), lambda i: (0, 0)),
        pl.BlockSpec((1, D_H), lambda i: (0, 0)),
        pl.BlockSpec((1, D_H), lambda i: (0, 0)),
        pl.BlockSpec((1, D_H), lambda i: (0, 0)),
    ],
    out_specs=[pl.BlockSpec((_R, D_H), lambda i: (i, 0))] * 2,
    out_shape=[jax.ShapeDtypeStruct((N, D_H), jnp.float32)] * 2,
)


def _tc_out_body(a_ref, h_ref, deg_ref, b_ref, pw1_ref, pb1_ref,
                 pw2_ref, pb2_ref, emb_ref, o_ref):
    dinv = lax.rsqrt(deg_ref[...] + 1.0)
    z = dinv * a_ref[...] + (dinv * dinv) * h_ref[...] + b_ref[...]
    emb_ref[...] = z
    f = jnp.maximum(z, 0.0)
    y = jnp.dot(f, pw1_ref[...], preferred_element_type=jnp.float32) + pb1_ref[...]
    y = jnp.dot(y, pw2_ref[...], preferred_element_type=jnp.float32) + pb2_ref[...]
    m = jnp.max(y, axis=-1, keepdims=True)
    lse = jnp.log(jnp.sum(jnp.exp(y - m), axis=-1, keepdims=True)) + m
    o_ref[...] = y - lse


_tc_out = pl.pallas_call(
    _tc_out_body,
    grid=(_GRID,),
    in_specs=[
        pl.BlockSpec((_R, D_H), lambda i: (i, 0)),
        pl.BlockSpec((_R, D_H), lambda i: (i, 0)),
        pl.BlockSpec((_R, 1), lambda i: (i, 0)),
        pl.BlockSpec((1, D_H), lambda i: (0, 0)),
        pl.BlockSpec((D_H, D_H), lambda i: (0, 0)),
        pl.BlockSpec((1, D_H), lambda i: (0, 0)),
        pl.BlockSpec((D_H, D_OUT), lambda i: (0, 0)),
        pl.BlockSpec((1, D_OUT), lambda i: (0, 0)),
    ],
    out_specs=[
        pl.BlockSpec((_R, D_H), lambda i: (i, 0)),
        pl.BlockSpec((_R, D_OUT), lambda i: (i, 0)),
    ],
    out_shape=[
        jax.ShapeDtypeStruct((N, D_H), jnp.float32),
        jax.ShapeDtypeStruct((N, D_OUT), jnp.float32),
    ],
)


@jax.jit
def kernel(x, edge_index, W1, b1, W2, b2, W3, b3, g1, be1, g2, be2,
           pw1, pb1, pw2, pb2):
    src = edge_index[0]
    dst = edge_index[1]
    pad = E_PAD - E
    src2 = jnp.pad(src, (0, pad)).reshape(ROWS2D, BATCH)
    dst2 = jnp.pad(dst, (0, pad), constant_values=N).reshape(ROWS2D, BATCH)
    zeros2 = jnp.zeros((QR, D_H), jnp.float32)
    zeros1 = jnp.zeros((DR,), jnp.float32)

    _sc_deg = _sc_deg_kernel()
    _sc_agg = _sc_agg_kernel()
    deg2 = _sc_deg(dst2, zeros1)
    deg = jnp.concatenate([deg2[:HALF], deg2[DR:DR + HALF]])[:, None]

    b1r, b2r, b3r = b1[None, :], b2[None, :], b3[None, :]
    g1r, be1r = g1[None, :], be1[None, :]
    g2r, be2r = g2[None, :], be2[None, :]
    pb1r, pb2r = pb1[None, :], pb2[None, :]

    def agg(p):
        a = _sc_agg(p, src2, dst2, zeros2)
        return jnp.concatenate(
            [a[r * QR:r * QR + QUARTER] for r in range(NQ)], axis=0)

    h1, p1 = _tc_prep(x, deg, W1)
    a1 = agg(p1)
    h2, p2 = _tc_mid(a1, h1, deg, W2, b1r, g1r, be1r)
    a2 = agg(p2)
    h3, p3 = _tc_mid(a2, h2, deg, W3, b2r, g2r, be2r)
    a3 = agg(p3)
    emb, out2 = _tc_out(a3, h3, deg, b3r, pw1, pb1r, pw2, pb2r)
    return emb, out2
